# Initial kernel scaffold; baseline (speedup 1.0000x reference)
import jax
import jax.numpy as jnp
from jax.experimental import pallas as pl


def kernel(x, edge_index, W1l, b1l, W1r, W2l, b2l, W2r):
    raise NotImplementedError("write your pallas kernel here")


# speedup vs baseline: 5.5060x; 5.5060x over previous
import functools

import jax
import jax.numpy as jnp
from jax import lax
from jax.experimental import pallas as pl
from jax.experimental.pallas import tpu as pltpu
from jax.experimental.pallas import tpu_sc as plsc

NC = 2
NS = 16
NW = NC * NS
N = 10000
D = 128
E = 320000
EPW = E // NW
CH = 80
NCHUNK = EPW // CH
RPT = 624
CR = 48
TAIL = NS * RPT
CNTW = 16


def _sc_body_with_cnt(x_hbm, src_hbm, dst_hbm, aggp, cntp,
                      agg_sh, cnt_sh, srcb, dstb, rows, zbuf, zcbuf, ones,
                      sem):
    _sc_common(x_hbm, src_hbm, dst_hbm, aggp, cntp,
               agg_sh, cnt_sh, srcb, dstb, rows, zbuf, zcbuf, ones, sem)


def _sc_body_no_cnt(x_hbm, src_hbm, dst_hbm, aggp,
                    agg_sh, srcb, dstb, rows, zbuf, sem):
    _sc_common(x_hbm, src_hbm, dst_hbm, aggp, None,
               agg_sh, None, srcb, dstb, rows, zbuf, None, None, sem)


def _sc_common(x_hbm, src_hbm, dst_hbm, aggp, cntp,
               agg_sh, cnt_sh, srcb, dstb, rows, zbuf, zcbuf, ones, sem):
    c = lax.axis_index("c")
    s = lax.axis_index("s")
    wid = c * NS + s
    start = s * RPT

    zero16 = jnp.zeros((16,), jnp.float32)

    def zrow(r, _):
        def zcol(k, _):
            zbuf[r, pl.ds(k * 16, 16)] = zero16
            return 0
        return lax.fori_loop(0, D // 16, zcol, 0)
    lax.fori_loop(0, CR, zrow, 0)

    if ones is not None:
        one16 = jnp.ones((16,), jnp.float32)

        def orow(r, _):
            ones[r, :] = one16
            return 0
        lax.fori_loop(0, CH, orow, 0)

        def zcrow(r, _):
            zcbuf[r, :] = zero16
            return 0
        lax.fori_loop(0, CR, zcrow, 0)

    for k in range(RPT // CR):
        pltpu.sync_copy(zbuf, agg_sh.at[pl.ds(start + k * CR, CR)])
        if cnt_sh is not None:
            pltpu.sync_copy(zcbuf, cnt_sh.at[pl.ds(start + k * CR, CR)])

    @pl.when(s == NS - 1)
    def _ztail():
        pltpu.sync_copy(zbuf.at[pl.ds(0, N - TAIL)],
                        agg_sh.at[pl.ds(TAIL, N - TAIL)])
        if cnt_sh is not None:
            pltpu.sync_copy(zcbuf.at[pl.ds(0, N - TAIL)],
                            cnt_sh.at[pl.ds(TAIL, N - TAIL)])

    plsc.subcore_barrier()

    ebase = wid * EPW

    def chunk(j, _):
        pltpu.sync_copy(src_hbm.at[pl.ds(ebase + j * CH, CH)], srcb)
        pltpu.sync_copy(dst_hbm.at[pl.ds(ebase + j * CH, CH)], dstb)
        pltpu.async_copy(x_hbm.at[srcb], rows, sem).wait()
        pltpu.sync_copy(rows, agg_sh.at[dstb], add=True)
        if cnt_sh is not None:
            pltpu.sync_copy(ones, cnt_sh.at[dstb], add=True)
        return 0
    lax.fori_loop(0, NCHUNK, chunk, 0)

    plsc.subcore_barrier()

    def copy_rows(off, nrows):
        pltpu.sync_copy(agg_sh.at[pl.ds(off, nrows)], zbuf.at[pl.ds(0, nrows)])
        pltpu.sync_copy(zbuf.at[pl.ds(0, nrows)], aggp.at[c, pl.ds(off, nrows)])
        if cnt_sh is not None:
            pltpu.sync_copy(cnt_sh.at[pl.ds(off, nrows)],
                            zcbuf.at[pl.ds(0, nrows)])
            pltpu.sync_copy(zcbuf.at[pl.ds(0, nrows)],
                            cntp.at[c, pl.ds(off, nrows)])

    for k in range(RPT // CR):
        copy_rows(start + k * CR, CR)

    @pl.when(s == NS - 1)
    def _ctail():
        copy_rows(TAIL, N - TAIL)


@functools.cache
def _sc_agg(with_cnt: bool):
    mesh = plsc.VectorSubcoreMesh(core_axis_name="c", subcore_axis_name="s",
                                  num_cores=NC, num_subcores=NS)
    out_type = [jax.ShapeDtypeStruct((NC, N, D), jnp.float32)]
    scratch = [
        pltpu.VMEM_SHARED((N, D), jnp.float32),
    ]
    if with_cnt:
        out_type.append(jax.ShapeDtypeStruct((NC, N, CNTW), jnp.float32))
        scratch.append(pltpu.VMEM_SHARED((N, CNTW), jnp.float32))
    scratch += [
        pltpu.VMEM((CH,), jnp.int32),
        pltpu.VMEM((CH,), jnp.int32),
        pltpu.VMEM((CH, D), jnp.float32),
        pltpu.VMEM((CR, D), jnp.float32),
    ]
    if with_cnt:
        scratch.append(pltpu.VMEM((CR, CNTW), jnp.float32))
        scratch.append(pltpu.VMEM((CH, CNTW), jnp.float32))
    scratch.append(pltpu.SemaphoreType.DMA)
    body = _sc_body_with_cnt if with_cnt else _sc_body_no_cnt
    return pl.kernel(body, out_type=tuple(out_type), mesh=mesh,
                     scratch_types=tuple(scratch),
                     compiler_params=pltpu.CompilerParams(
                         use_tc_tiling_on_sc=False))


def _tc_layer_body(act, aggp_ref, cntp_ref, x_ref, wl_ref, bl_ref, wr_ref,
                   o_ref):
    agg = aggp_ref[0] + aggp_ref[1]
    cnt = cntp_ref[0, :, 0:1] + cntp_ref[1, :, 0:1]
    mean = agg / jnp.maximum(cnt, 1.0)
    out = jnp.dot(mean, wl_ref[...], preferred_element_type=jnp.float32)
    out = out + bl_ref[...]
    out = out + jnp.dot(x_ref[...], wr_ref[...],
                        preferred_element_type=jnp.float32)
    if act == "elu":
        o_ref[...] = jnp.where(out > 0, out,
                               jnp.exp(jnp.minimum(out, 0.0)) - 1.0)
    else:
        m = jnp.max(out, axis=1, keepdims=True)
        lse = jnp.log(jnp.sum(jnp.exp(out - m), axis=1, keepdims=True)) + m
        o_ref[...] = out - lse


@functools.cache
def _tc_layer(act: str):
    BR = 1000
    grid = (N // BR,)
    return pl.pallas_call(
        functools.partial(_tc_layer_body, act),
        grid=grid,
        in_specs=[
            pl.BlockSpec((NC, BR, D), lambda i: (0, i, 0)),
            pl.BlockSpec((NC, BR, CNTW), lambda i: (0, i, 0)),
            pl.BlockSpec((BR, D), lambda i: (i, 0)),
            pl.BlockSpec((D, D), lambda i: (0, 0)),
            pl.BlockSpec((1, D), lambda i: (0, 0)),
            pl.BlockSpec((D, D), lambda i: (0, 0)),
        ],
        out_specs=pl.BlockSpec((BR, D), lambda i: (i, 0)),
        out_shape=jax.ShapeDtypeStruct((N, D), jnp.float32),
    )


@jax.jit
def kernel(x, edge_index, W1l, b1l, W1r, W2l, b2l, W2r):
    src = edge_index[0].astype(jnp.int32)
    dst = edge_index[1].astype(jnp.int32)
    aggp1, cntp = _sc_agg(True)(x, src, dst)
    h = _tc_layer("elu")(aggp1, cntp, x, W1l.T, b1l.reshape(1, D), W1r.T)
    aggp2, = _sc_agg(False)(h, src, dst)
    return _tc_layer("lsm")(aggp2, cntp, h, W2l.T, b2l.reshape(1, D), W2r.T)


# speedup vs baseline: 12.0062x; 2.1806x over previous
import functools

import jax
import jax.numpy as jnp
from jax import lax
from jax.experimental import pallas as pl
from jax.experimental.pallas import tpu as pltpu
from jax.experimental.pallas import tpu_sc as plsc

NC = 2
NS = 16
NW = NC * NS
N = 10000
D = 128
E = 320000
EPW = E // NW
CH = 80
NCHUNK = EPW // CH
KB = 25
RPT = 624
CR = 48
TAIL = NS * RPT
CNTW = 16


def _sc_body_with_cnt(x_hbm, src_hbm, dst_hbm, aggp, cntp,
                      agg_sh, cnt_sh, srcb, dstb, rows, zbuf, zcbuf, ones,
                      sem):
    _sc_common(x_hbm, src_hbm, dst_hbm, aggp, cntp,
               agg_sh, cnt_sh, srcb, dstb, rows, zbuf, zcbuf, ones, sem)


def _sc_body_no_cnt(x_hbm, src_hbm, dst_hbm, aggp,
                    agg_sh, srcb, dstb, rows, zbuf, sem):
    _sc_common(x_hbm, src_hbm, dst_hbm, aggp, None,
               agg_sh, None, srcb, dstb, rows, zbuf, None, None, sem)


def _sc_common(x_hbm, src_hbm, dst_hbm, aggp, cntp,
               agg_sh, cnt_sh, srcb, dstb, rows, zbuf, zcbuf, ones, sem):
    c = lax.axis_index("c")
    s = lax.axis_index("s")
    wid = c * NS + s
    start = s * RPT

    zero16 = jnp.zeros((16,), jnp.float32)

    def zrow(r, _):
        def zcol(k, _):
            zbuf[r, pl.ds(k * 16, 16)] = zero16
            return 0
        return lax.fori_loop(0, D // 16, zcol, 0)
    lax.fori_loop(0, CR, zrow, 0)

    if ones is not None:
        one16 = jnp.ones((16,), jnp.float32)

        def orow(r, _):
            ones[r, :] = one16
            return 0
        lax.fori_loop(0, CH, orow, 0)

        def zcrow(r, _):
            zcbuf[r, :] = zero16
            return 0
        lax.fori_loop(0, CR, zcrow, 0)

    for k in range(RPT // CR):
        pltpu.sync_copy(zbuf, agg_sh.at[pl.ds(start + k * CR, CR)])
        if cnt_sh is not None:
            pltpu.sync_copy(zcbuf, cnt_sh.at[pl.ds(start + k * CR, CR)])

    @pl.when(s == NS - 1)
    def _ztail():
        pltpu.sync_copy(zbuf.at[pl.ds(0, N - TAIL)],
                        agg_sh.at[pl.ds(TAIL, N - TAIL)])
        if cnt_sh is not None:
            pltpu.sync_copy(zcbuf.at[pl.ds(0, N - TAIL)],
                            cnt_sh.at[pl.ds(TAIL, N - TAIL)])

    plsc.subcore_barrier()

    rbase = wid * NCHUNK

    def load_bulk(b, bb):
        pltpu.sync_copy(src_hbm.at[pl.ds(rbase + b * KB, KB)], srcb.at[bb])
        pltpu.sync_copy(dst_hbm.at[pl.ds(rbase + b * KB, KB)], dstb.at[bb])

    def fire(j):
        bb = (j // KB) % 2
        pltpu.async_copy(x_hbm.at[srcb.at[bb, j % KB]], rows.at[j % 2],
                         sem.at[j % 2])

    load_bulk(0, 0)
    fire(0)

    def chunk(j, _):
        nj = j + 1

        @pl.when(jnp.logical_and(nj % KB == 0, nj < NCHUNK))
        def _load():
            load_bulk(nj // KB, (nj // KB) % 2)

        @pl.when(nj < NCHUNK)
        def _fire():
            fire(nj)

        bb = (j // KB) % 2
        pltpu.make_async_copy(x_hbm.at[pl.ds(0, CH)], rows.at[j % 2],
                              sem.at[j % 2]).wait()
        pltpu.sync_copy(rows.at[j % 2], agg_sh.at[dstb.at[bb, j % KB]],
                        add=True)
        if cnt_sh is not None:
            pltpu.sync_copy(ones, cnt_sh.at[dstb.at[bb, j % KB]], add=True)
        return 0
    lax.fori_loop(0, NCHUNK, chunk, 0)

    plsc.subcore_barrier()

    def copy_rows(off, nrows):
        pltpu.sync_copy(agg_sh.at[pl.ds(off, nrows)], zbuf.at[pl.ds(0, nrows)])
        pltpu.sync_copy(zbuf.at[pl.ds(0, nrows)], aggp.at[c, pl.ds(off, nrows)])
        if cnt_sh is not None:
            pltpu.sync_copy(cnt_sh.at[pl.ds(off, nrows)],
                            zcbuf.at[pl.ds(0, nrows)])
            pltpu.sync_copy(zcbuf.at[pl.ds(0, nrows)],
                            cntp.at[c, pl.ds(off, nrows)])

    for k in range(RPT // CR):
        copy_rows(start + k * CR, CR)

    @pl.when(s == NS - 1)
    def _ctail():
        copy_rows(TAIL, N - TAIL)


@functools.cache
def _sc_agg(with_cnt: bool):
    mesh = plsc.VectorSubcoreMesh(core_axis_name="c", subcore_axis_name="s",
                                  num_cores=NC, num_subcores=NS)
    out_type = [jax.ShapeDtypeStruct((NC, N, D), jnp.float32)]
    scratch = [
        pltpu.VMEM_SHARED((N, D), jnp.float32),
    ]
    if with_cnt:
        out_type.append(jax.ShapeDtypeStruct((NC, N, CNTW), jnp.float32))
        scratch.append(pltpu.VMEM_SHARED((N, CNTW), jnp.float32))
    scratch += [
        pltpu.VMEM((2, KB, CH), jnp.int32),
        pltpu.VMEM((2, KB, CH), jnp.int32),
        pltpu.VMEM((2, CH, D), jnp.float32),
        pltpu.VMEM((CR, D), jnp.float32),
    ]
    if with_cnt:
        scratch.append(pltpu.VMEM((CR, CNTW), jnp.float32))
        scratch.append(pltpu.VMEM((CH, CNTW), jnp.float32))
    scratch.append(pltpu.SemaphoreType.DMA((2,)))
    body = _sc_body_with_cnt if with_cnt else _sc_body_no_cnt
    return pl.kernel(body, out_type=tuple(out_type), mesh=mesh,
                     scratch_types=tuple(scratch),
                     compiler_params=pltpu.CompilerParams(
                         use_tc_tiling_on_sc=False))


def _tc_layer_body(act, aggp_ref, cntp_ref, x_ref, wl_ref, bl_ref, wr_ref,
                   o_ref):
    agg = aggp_ref[0] + aggp_ref[1]
    cnt = cntp_ref[0, :, 0:1] + cntp_ref[1, :, 0:1]
    mean = agg / jnp.maximum(cnt, 1.0)
    out = jnp.dot(mean, wl_ref[...], preferred_element_type=jnp.float32)
    out = out + bl_ref[...]
    out = out + jnp.dot(x_ref[...], wr_ref[...],
                        preferred_element_type=jnp.float32)
    if act == "elu":
        o_ref[...] = jnp.where(out > 0, out,
                               jnp.exp(jnp.minimum(out, 0.0)) - 1.0)
    else:
        m = jnp.max(out, axis=1, keepdims=True)
        lse = jnp.log(jnp.sum(jnp.exp(out - m), axis=1, keepdims=True)) + m
        o_ref[...] = out - lse


@functools.cache
def _tc_layer(act: str):
    BR = 1000
    grid = (N // BR,)
    return pl.pallas_call(
        functools.partial(_tc_layer_body, act),
        grid=grid,
        in_specs=[
            pl.BlockSpec((NC, BR, D), lambda i: (0, i, 0)),
            pl.BlockSpec((NC, BR, CNTW), lambda i: (0, i, 0)),
            pl.BlockSpec((BR, D), lambda i: (i, 0)),
            pl.BlockSpec((D, D), lambda i: (0, 0)),
            pl.BlockSpec((1, D), lambda i: (0, 0)),
            pl.BlockSpec((D, D), lambda i: (0, 0)),
        ],
        out_specs=pl.BlockSpec((BR, D), lambda i: (i, 0)),
        out_shape=jax.ShapeDtypeStruct((N, D), jnp.float32),
    )


@jax.jit
def kernel(x, edge_index, W1l, b1l, W1r, W2l, b2l, W2r):
    src = edge_index[0].astype(jnp.int32).reshape(E // CH, CH)
    dst = edge_index[1].astype(jnp.int32).reshape(E // CH, CH)
    aggp1, cntp = _sc_agg(True)(x, src, dst)
    h = _tc_layer("elu")(aggp1, cntp, x, W1l.T, b1l.reshape(1, D), W1r.T)
    aggp2, = _sc_agg(False)(h, src, dst)
    return _tc_layer("lsm")(aggp2, cntp, h, W2l.T, b2l.reshape(1, D), W2r.T)


# speedup vs baseline: 12.4246x; 1.0349x over previous
import functools

import jax
import jax.numpy as jnp
from jax import lax
from jax.experimental import pallas as pl
from jax.experimental.pallas import tpu as pltpu
from jax.experimental.pallas import tpu_sc as plsc

NC = 2
NS = 16
NW = NC * NS
N = 10000
D = 128
E = 320000
RPT = 624
CR = 48
TAIL = NS * RPT
CNTW = 16

CH1, KB1, DEPTH1 = 80, 25, 2
CH2, KB2, DEPTH2 = 128, 13, 2


def _sc_common(with_cnt, ch, kb, depth,
               x_hbm, src_hbm, dst_hbm, aggp, cntp,
               agg_sh, cnt_sh, srcb, dstb, rows, zbuf, zcbuf, ones, sem):
    c = lax.axis_index("c")
    s = lax.axis_index("s")
    wid = c * NS + s
    start = s * RPT

    zero16 = jnp.zeros((16,), jnp.float32)

    def zrow(r, _):
        def zcol(k, _):
            zbuf[r, pl.ds(k * 16, 16)] = zero16
            return 0
        return lax.fori_loop(0, D // 16, zcol, 0)
    lax.fori_loop(0, CR, zrow, 0)

    if with_cnt:
        one16 = jnp.ones((16,), jnp.float32)

        def orow(r, _):
            ones[r, :] = one16
            return 0
        lax.fori_loop(0, ch, orow, 0)

        def zcrow(r, _):
            zcbuf[r, :] = zero16
            return 0
        lax.fori_loop(0, CR, zcrow, 0)

    for k in range(RPT // CR):
        pltpu.sync_copy(zbuf, agg_sh.at[pl.ds(start + k * CR, CR)])
        if with_cnt:
            pltpu.sync_copy(zcbuf, cnt_sh.at[pl.ds(start + k * CR, CR)])

    @pl.when(s == NS - 1)
    def _ztail():
        pltpu.sync_copy(zbuf.at[pl.ds(0, N - TAIL)],
                        agg_sh.at[pl.ds(TAIL, N - TAIL)])
        if with_cnt:
            pltpu.sync_copy(zcbuf.at[pl.ds(0, N - TAIL)],
                            cnt_sh.at[pl.ds(TAIL, N - TAIL)])

    plsc.subcore_barrier()

    ntotal = E // ch
    base = ntotal // NW
    rem = ntotal % NW
    cw = base + jnp.where(wid < rem, 1, 0)
    rbase = wid * base + jnp.minimum(wid, rem)

    def load_bulk(b):
        bb = b % 2
        pltpu.sync_copy(src_hbm.at[pl.ds(rbase + b * kb, kb)], srcb.at[bb])
        pltpu.sync_copy(dst_hbm.at[pl.ds(rbase + b * kb, kb)], dstb.at[bb])

    def fire(j):
        bb = (j // kb) % 2
        pltpu.async_copy(x_hbm.at[srcb.at[bb, j % kb]], rows.at[j % depth],
                         sem.at[j % depth])

    load_bulk(0)
    fire(0)
    for k in range(1, depth - 1):
        @pl.when(k < cw)
        def _prefire(k=k):
            fire(k)

    def chunk(j, _):
        nj = j + depth - 1

        @pl.when(jnp.logical_and(nj % kb == 0, nj < cw))
        def _load():
            load_bulk(nj // kb)

        @pl.when(nj < cw)
        def _fire():
            fire(nj)

        bb = (j // kb) % 2
        pltpu.make_async_copy(x_hbm.at[pl.ds(0, ch)], rows.at[j % depth],
                              sem.at[j % depth]).wait()
        pltpu.sync_copy(rows.at[j % depth], agg_sh.at[dstb.at[bb, j % kb]],
                        add=True)
        if with_cnt:
            pltpu.sync_copy(ones, cnt_sh.at[dstb.at[bb, j % kb]], add=True)
        return 0
    lax.fori_loop(0, cw, chunk, 0)

    plsc.subcore_barrier()

    def copy_rows(off, nrows):
        pltpu.sync_copy(agg_sh.at[pl.ds(off, nrows)], zbuf.at[pl.ds(0, nrows)])
        pltpu.sync_copy(zbuf.at[pl.ds(0, nrows)], aggp.at[c, pl.ds(off, nrows)])
        if with_cnt:
            pltpu.sync_copy(cnt_sh.at[pl.ds(off, nrows)],
                            zcbuf.at[pl.ds(0, nrows)])
            pltpu.sync_copy(zcbuf.at[pl.ds(0, nrows)],
                            cntp.at[c, pl.ds(off, nrows)])

    for k in range(RPT // CR):
        copy_rows(start + k * CR, CR)

    @pl.when(s == NS - 1)
    def _ctail():
        copy_rows(TAIL, N - TAIL)


@functools.cache
def _sc_agg(with_cnt: bool, ch: int, kb: int, depth: int):
    mesh = plsc.VectorSubcoreMesh(core_axis_name="c", subcore_axis_name="s",
                                  num_cores=NC, num_subcores=NS)
    out_type = [jax.ShapeDtypeStruct((NC, N, D), jnp.float32)]
    scratch = [
        pltpu.VMEM_SHARED((N, D), jnp.float32),
    ]
    if with_cnt:
        out_type.append(jax.ShapeDtypeStruct((NC, N, CNTW), jnp.float32))
        scratch.append(pltpu.VMEM_SHARED((N, CNTW), jnp.float32))
    scratch += [
        pltpu.VMEM((2, kb, ch), jnp.int32),
        pltpu.VMEM((2, kb, ch), jnp.int32),
        pltpu.VMEM((depth, ch, D), jnp.float32),
        pltpu.VMEM((CR, D), jnp.float32),
    ]
    if with_cnt:
        scratch.append(pltpu.VMEM((CR, CNTW), jnp.float32))
        scratch.append(pltpu.VMEM((ch, CNTW), jnp.float32))
    scratch.append(pltpu.SemaphoreType.DMA((depth,)))

    if with_cnt:
        def body(x_hbm, src_hbm, dst_hbm, aggp, cntp,
                 agg_sh, cnt_sh, srcb, dstb, rows, zbuf, zcbuf, ones, sem):
            _sc_common(True, ch, kb, depth,
                       x_hbm, src_hbm, dst_hbm, aggp, cntp,
                       agg_sh, cnt_sh, srcb, dstb, rows, zbuf, zcbuf, ones,
                       sem)
    else:
        def body(x_hbm, src_hbm, dst_hbm, aggp,
                 agg_sh, srcb, dstb, rows, zbuf, sem):
            _sc_common(False, ch, kb, depth,
                       x_hbm, src_hbm, dst_hbm, aggp, None,
                       agg_sh, None, srcb, dstb, rows, zbuf, None, None, sem)

    return pl.kernel(body, out_type=tuple(out_type), mesh=mesh,
                     scratch_types=tuple(scratch),
                     compiler_params=pltpu.CompilerParams(
                         use_tc_tiling_on_sc=False))


def _tc_layer_body(act, aggp_ref, cntp_ref, x_ref, wl_ref, bl_ref, wr_ref,
                   o_ref):
    agg = aggp_ref[0] + aggp_ref[1]
    cnt = cntp_ref[0, :, 0:1] + cntp_ref[1, :, 0:1]
    mean = agg / jnp.maximum(cnt, 1.0)
    out = jnp.dot(mean, wl_ref[...], preferred_element_type=jnp.float32)
    out = out + bl_ref[...]
    out = out + jnp.dot(x_ref[...], wr_ref[...],
                        preferred_element_type=jnp.float32)
    if act == "elu":
        o_ref[...] = jnp.where(out > 0, out,
                               jnp.exp(jnp.minimum(out, 0.0)) - 1.0)
    else:
        m = jnp.max(out, axis=1, keepdims=True)
        lse = jnp.log(jnp.sum(jnp.exp(out - m), axis=1, keepdims=True)) + m
        o_ref[...] = out - lse


@functools.cache
def _tc_layer(act: str):
    BR = 1000
    grid = (N // BR,)
    return pl.pallas_call(
        functools.partial(_tc_layer_body, act),
        grid=grid,
        in_specs=[
            pl.BlockSpec((NC, BR, D), lambda i: (0, i, 0)),
            pl.BlockSpec((NC, BR, CNTW), lambda i: (0, i, 0)),
            pl.BlockSpec((BR, D), lambda i: (i, 0)),
            pl.BlockSpec((D, D), lambda i: (0, 0)),
            pl.BlockSpec((1, D), lambda i: (0, 0)),
            pl.BlockSpec((D, D), lambda i: (0, 0)),
        ],
        out_specs=pl.BlockSpec((BR, D), lambda i: (i, 0)),
        out_shape=jax.ShapeDtypeStruct((N, D), jnp.float32),
    )


@jax.jit
def kernel(x, edge_index, W1l, b1l, W1r, W2l, b2l, W2r):
    src = edge_index[0].astype(jnp.int32)
    dst = edge_index[1].astype(jnp.int32)
    src1 = src.reshape(E // CH1, CH1)
    dst1 = dst.reshape(E // CH1, CH1)
    src2 = src.reshape(E // CH2, CH2)
    dst2 = dst.reshape(E // CH2, CH2)
    aggp1, cntp = _sc_agg(True, CH1, KB1, DEPTH1)(x, src1, dst1)
    h = _tc_layer("elu")(aggp1, cntp, x, W1l.T, b1l.reshape(1, D), W1r.T)
    aggp2, = _sc_agg(False, CH2, KB2, DEPTH2)(h, src2, dst2)
    return _tc_layer("lsm")(aggp2, cntp, h, W2l.T, b2l.reshape(1, D), W2r.T)
